# baseline (speedup 1.0000x reference)
import functools

import numpy as np
import jax
import jax.numpy as jnp
from jax import lax
from jax.experimental import pallas as pl
from jax.experimental.pallas import tpu as pltpu
from jax.experimental.pallas import tpu_sc as plsc

N = 10000
E = 320000
D = 128
NC = 2
NS = 16
K = 80
U = 6
CHUNKS = 252
CPAD = CHUNKS + U
RPT = 624
BM = 1000

_PERM = np.empty((D,), np.int32)
for _j in range(D // 32):
    for _i in range(16):
        _PERM[32 * _j + 2 * _i] = 32 * _j + _i
        _PERM[32 * _j + 2 * _i + 1] = 32 * _j + 16 + _i


def _matmul_body(x_ref, w_ref, o_ref):
    o_ref[0] = jnp.dot(x_ref[...], w_ref[0],
                       preferred_element_type=jnp.float32)


_matmul = pl.pallas_call(
    _matmul_body,
    grid=(2, N // BM),
    in_specs=[
        pl.BlockSpec((BM, D), lambda g, i: (i, 0)),
        pl.BlockSpec((1, D, D), lambda g, i: (g, 0, 0)),
    ],
    out_specs=pl.BlockSpec((1, BM, D), lambda g, i: (g, i, 0)),
    out_shape=jax.ShapeDtypeStruct((2, N, D), jnp.float32),
)


_sc_mesh = plsc.VectorSubcoreMesh(core_axis_name="c", subcore_axis_name="s")


@functools.partial(
    pl.kernel,
    out_type=jax.ShapeDtypeStruct((NC, N, D), jnp.float32),
    mesh=_sc_mesh,
    scratch_types=[
        pltpu.VMEM((U, 2, K), jnp.int32),
        pltpu.VMEM((U, 1, K), jnp.float32),
        pltpu.VMEM((K, D), jnp.float32),
        pltpu.VMEM((K, D), jnp.float32),
        pltpu.VMEM((K, D), jnp.float32),
        pltpu.VMEM((K, D), jnp.float32),
        pltpu.VMEM_SHARED((N, D), jnp.float32),
        pltpu.SemaphoreType.DMA,
        pltpu.SemaphoreType.DMA,
        pltpu.SemaphoreType.DMA,
        pltpu.SemaphoreType.DMA,
        pltpu.SemaphoreType.DMA,
        pltpu.SemaphoreType.DMA,
    ],
)
def _spmm_kernel(sup_hbm, idx_hbm, vals_hbm, out_hbm,
                 ibuf, vbuf, g0, g1, s0, s1, acc,
                 gsem0, gsem1, ssem0, ssem1, isem0, isem1):
    c = lax.axis_index("c")
    s = lax.axis_index("s")
    gbufs = (g0, g1)
    sbufs = (s0, s1)
    gsems = (gsem0, gsem1)
    ssems = (ssem0, ssem1)
    isems = (isem0, isem1)

    def iload(t, q):
        pltpu.async_copy(idx_hbm.at[c, s, t], ibuf.at[q], isems[q % 2])
        pltpu.async_copy(vals_hbm.at[c, s, t], vbuf.at[q], isems[q % 2])

    def iload_wait(q):
        pltpu.make_async_copy(idx_hbm.at[c, s, 0], ibuf.at[q],
                              isems[q % 2]).wait()
        pltpu.make_async_copy(vals_hbm.at[c, s, 0], vbuf.at[q],
                              isems[q % 2]).wait()

    def gather(q, p):
        pltpu.async_copy(sup_hbm.at[c].at[ibuf.at[q, 1]], gbufs[p], gsems[p])

    def gather_wait(p):
        pltpu.make_async_copy(sup_hbm.at[c, pl.ds(0, K), :], gbufs[p],
                              gsems[p]).wait()

    def scat(q, p):
        pltpu.async_copy(sbufs[p], acc.at[ibuf.at[q, 0]], ssems[p], add=True)

    def scat_wait(p):
        pltpu.make_async_copy(sbufs[p], acc.at[pl.ds(0, K), :],
                              ssems[p]).wait()

    def scale(q, p):
        g = gbufs[p]
        sb = sbufs[p]

        @plsc.parallel_loop(0, K // 16)
        def _sbody(gr):
            vv = vbuf[q, 0, pl.ds(gr * 16, 16)]
            for l in range(16):
                v = vv[l]
                i = gr * 16 + l
                for jj in range(D // 16):
                    sl = pl.ds(jj * 16, 16)
                    sb[i, sl] = g[i, sl] * v

    def step(t, j, skip_scat_wait=False):
        p = j % 2
        o = 1 - p
        iload_wait((j + 1) % U)
        gather((j + 1) % U, o)
        iload(t + 3, (j + 3) % U)
        gather_wait(p)
        if not skip_scat_wait:
            scat_wait(p)
        scale(j, p)
        scat(j, p)

    zero16 = jnp.zeros((16,), jnp.float32)

    def zbody(i, _):
        for j in range(D // 16):
            s0[i, pl.ds(j * 16, 16)] = zero16
        return 0

    lax.fori_loop(0, K, zbody, 0)
    base = s * RPT
    for t in range(RPT // K):
        pltpu.sync_copy(s0, acc.at[pl.ds(base + t * K, K), :])
    if RPT % K:
        pltpu.sync_copy(s0.at[pl.ds(0, RPT % K), :],
                        acc.at[pl.ds(base + (RPT // K) * K, RPT % K), :])

    @pl.when(s == NS - 1)
    def _zero_tail():
        pltpu.sync_copy(s0.at[pl.ds(0, N - NS * RPT), :],
                        acc.at[pl.ds(NS * RPT, N - NS * RPT), :])

    plsc.subcore_barrier()

    iload(0, 0)
    iload_wait(0)
    gather(0, 0)
    iload(1, 1)
    iload(2, 2)

    step(0, 0, skip_scat_wait=True)
    step(1, 1, skip_scat_wait=True)
    step(2, 2)
    step(3, 3)
    step(4, 4)
    step(5, 5)

    def hexa(u, _):
        t = u * U
        for j in range(U):
            step(t + j, j)
        return 0

    lax.fori_loop(1, CHUNKS // U, hexa, 0)

    scat_wait(0)
    scat_wait(1)
    gather_wait(0)
    iload_wait((CHUNKS + 1) % U)
    iload_wait((CHUNKS + 2) % U)

    plsc.subcore_barrier()
    pltpu.sync_copy(acc.at[pl.ds(base, RPT), :],
                    out_hbm.at[c, pl.ds(base, RPT), :])

    @pl.when(s == NS - 1)
    def _drain_tail():
        pltpu.sync_copy(acc.at[pl.ds(NS * RPT, N - NS * RPT), :],
                        out_hbm.at[c, pl.ds(NS * RPT, N - NS * RPT), :])


def _combine_body(p_ref, b_ref, o_ref):
    o_ref[...] = p_ref[0] + p_ref[1] + b_ref[...]


_combine = pl.pallas_call(
    _combine_body,
    grid=(N // BM,),
    in_specs=[
        pl.BlockSpec((2, BM, D), lambda i: (0, i, 0)),
        pl.BlockSpec((1, D), lambda i: (0, 0)),
    ],
    out_specs=pl.BlockSpec((BM, D), lambda i: (i, 0)),
    out_shape=jax.ShapeDtypeStruct((N, D), jnp.float32),
)


def _pad_rs(a):
    per_tile = E // NS
    a = a.reshape(NS, per_tile)
    a = jnp.pad(a, ((0, 0), (0, CPAD * K - per_tile)))
    return a.reshape(NS, CPAD, K)


def _prep_idx(rows, cols):
    return jnp.stack([_pad_rs(rows.astype(jnp.int32)),
                      _pad_rs(cols.astype(jnp.int32))], axis=2)


def _prep_val(vals):
    return _pad_rs(vals.astype(jnp.float32))[:, :, None, :]


def kernel(input, weight_1, weight_2, bias,
           adj0_rows, adj0_cols, adj0_vals,
           adj1_rows, adj1_cols, adj1_vals):
    w = jnp.stack([weight_1, weight_2])
    sup = _matmul(input, w)
    idx = jnp.stack([_prep_idx(adj0_rows, adj0_cols),
                     _prep_idx(adj1_rows, adj1_cols)])
    vals = jnp.stack([_prep_val(adj0_vals), _prep_val(adj1_vals)])
    partial = _spmm_kernel(sup, idx, vals)
    return _combine(partial, bias.reshape(1, D))
